# baseline (speedup 1.0000x reference)
import functools

import jax
import jax.numpy as jnp
from jax import lax
from jax.experimental import pallas as pl
from jax.experimental.pallas import tpu as pltpu
from jax.experimental.pallas import tpu_sc as plsc

HEADS = 8
OUT_CH = 16
FEAT = HEADS * OUT_CH
LANES = 16
NC = 2
NS = 16
NW = NC * NS
CHUNK = 112
ZROW = 64

_GD = lax.GatherDimensionNumbers(
    offset_dims=(), collapsed_slice_dims=(0,), start_index_map=(0,))


def _splat(v, h):
    idx = jnp.full((LANES, 1), h, dtype=jnp.int32)
    return lax.gather(v, idx, _GD, (1,),
                      mode=lax.GatherScatterMode.PROMISE_IN_BOUNDS)


def _build_a32(a_src, a_dst):
    rows = jnp.arange(FEAT)
    head = rows // OUT_CH
    a = jnp.zeros((FEAT, 2 * HEADS), jnp.float32)
    a = a.at[rows, head].set(a_src.reshape(-1))
    a = a.at[rows, HEADS + head].set(a_dst.reshape(-1))
    swap = jnp.concatenate([a[:, HEADS:], a[:, :HEADS]], axis=1)
    return jnp.concatenate([a, swap], axis=1)


def _tc_head_body(x_ref, w_ref, a_ref, h_ref, t1_ref, t2_ref):
    h = jnp.dot(x_ref[...], w_ref[...], preferred_element_type=jnp.float32)
    h_ref[...] = h
    t = jnp.dot(h, a_ref[...], preferred_element_type=jnp.float32)
    t1_ref[...] = t[:, :16]
    t2_ref[...] = t[:, 16:]


def _tc_mid_body(a0_ref, a1_ref, d0_ref, d1_ref, rep_ref, b_ref, w_ref,
                 a_ref, h_ref, t1_ref, t2_ref):
    dinv = 1.0 / (d0_ref[...] + d1_ref[...] + 1e-16)
    scale = jnp.dot(dinv, rep_ref[...], preferred_element_type=jnp.float32)
    x = jnp.maximum((a0_ref[...] + a1_ref[...]) * scale + b_ref[...], 0.0)
    h = jnp.dot(x, w_ref[...], preferred_element_type=jnp.float32)
    h_ref[...] = h
    t = jnp.dot(h, a_ref[...], preferred_element_type=jnp.float32)
    t1_ref[...] = t[:, :16]
    t2_ref[...] = t[:, 16:]


def _tc_tail_body(a0_ref, a1_ref, d0_ref, d1_ref, rep_ref, b_ref, o_ref):
    dinv = 1.0 / (d0_ref[...] + d1_ref[...] + 1e-16)
    scale = jnp.dot(dinv, rep_ref[...], preferred_element_type=jnp.float32)
    o_ref[...] = (a0_ref[...] + a1_ref[...]) * scale + b_ref[...]


def _tc_head(x_pad, w, a32, npad, br):
    grid = npad // br
    return pl.pallas_call(
        _tc_head_body,
        grid=(grid,),
        in_specs=[
            pl.BlockSpec((br, FEAT), lambda i: (i, 0)),
            pl.BlockSpec((FEAT, FEAT), lambda i: (0, 0)),
            pl.BlockSpec((FEAT, 32), lambda i: (0, 0)),
        ],
        out_specs=[
            pl.BlockSpec((br, FEAT), lambda i: (i, 0)),
            pl.BlockSpec((br, 16), lambda i: (i, 0)),
            pl.BlockSpec((br, 16), lambda i: (i, 0)),
        ],
        out_shape=[
            jax.ShapeDtypeStruct((npad, FEAT), jnp.float32),
            jax.ShapeDtypeStruct((npad, 16), jnp.float32),
            jax.ShapeDtypeStruct((npad, 16), jnp.float32),
        ],
    )(x_pad, w, a32)


def _tc_mid(acc0, acc1, d0, d1, rep, b, w, a32, npad, br):
    grid = npad // br
    return pl.pallas_call(
        _tc_mid_body,
        grid=(grid,),
        in_specs=[
            pl.BlockSpec((br, FEAT), lambda i: (i, 0)),
            pl.BlockSpec((br, FEAT), lambda i: (i, 0)),
            pl.BlockSpec((br, 16), lambda i: (i, 0)),
            pl.BlockSpec((br, 16), lambda i: (i, 0)),
            pl.BlockSpec((16, FEAT), lambda i: (0, 0)),
            pl.BlockSpec((1, FEAT), lambda i: (0, 0)),
            pl.BlockSpec((FEAT, FEAT), lambda i: (0, 0)),
            pl.BlockSpec((FEAT, 32), lambda i: (0, 0)),
        ],
        out_specs=[
            pl.BlockSpec((br, FEAT), lambda i: (i, 0)),
            pl.BlockSpec((br, 16), lambda i: (i, 0)),
            pl.BlockSpec((br, 16), lambda i: (i, 0)),
        ],
        out_shape=[
            jax.ShapeDtypeStruct((npad, FEAT), jnp.float32),
            jax.ShapeDtypeStruct((npad, 16), jnp.float32),
            jax.ShapeDtypeStruct((npad, 16), jnp.float32),
        ],
    )(acc0, acc1, d0, d1, rep, b.reshape(1, FEAT), w, a32)


def _tc_tail(acc0, acc1, d0, d1, rep, b, npad, br):
    grid = npad // br
    return pl.pallas_call(
        _tc_tail_body,
        grid=(grid,),
        in_specs=[
            pl.BlockSpec((br, FEAT), lambda i: (i, 0)),
            pl.BlockSpec((br, FEAT), lambda i: (i, 0)),
            pl.BlockSpec((br, 16), lambda i: (i, 0)),
            pl.BlockSpec((br, 16), lambda i: (i, 0)),
            pl.BlockSpec((16, FEAT), lambda i: (0, 0)),
            pl.BlockSpec((1, FEAT), lambda i: (0, 0)),
        ],
        out_specs=pl.BlockSpec((br, FEAT), lambda i: (i, 0)),
        out_shape=jax.ShapeDtypeStruct((npad, FEAT), jnp.float32),
    )(acc0, acc1, d0, d1, rep, b.reshape(1, FEAT))


def _sc_layer(sd, t1, t2, h, npad, e_pad, we, nchunks):
    rows_pt = npad // NS
    mesh = plsc.VectorSubcoreMesh(
        core_axis_name="c", subcore_axis_name="s", num_cores=NC,
        num_subcores=NS)

    @functools.partial(
        pl.kernel,
        out_type=[
            jax.ShapeDtypeStruct((npad, FEAT), jnp.float32),
            jax.ShapeDtypeStruct((npad, FEAT), jnp.float32),
            jax.ShapeDtypeStruct((npad, 16), jnp.float32),
            jax.ShapeDtypeStruct((npad, 16), jnp.float32),
        ],
        mesh=mesh,
        scratch_types=[
            pltpu.VMEM((2, CHUNK), jnp.int32),
            pltpu.VMEM((2, CHUNK), jnp.int32),
            pltpu.VMEM((CHUNK,), jnp.int32),
            pltpu.VMEM((CHUNK,), jnp.int32),
            pltpu.VMEM((CHUNK, 16), jnp.float32),
            pltpu.VMEM((CHUNK, 16), jnp.float32),
            pltpu.VMEM((CHUNK, 16), jnp.float32),
            pltpu.VMEM((CHUNK, 16), jnp.float32),
            pltpu.VMEM((CHUNK, FEAT), jnp.float32),
            pltpu.VMEM((CHUNK, FEAT), jnp.float32),
            pltpu.VMEM_SHARED((npad, FEAT), jnp.float32),
            pltpu.VMEM_SHARED((npad, 16), jnp.float32),
            pltpu.SemaphoreType.DMA,
            pltpu.SemaphoreType.DMA,
            pltpu.SemaphoreType.DMA,
            pltpu.SemaphoreType.DMA,
            pltpu.SemaphoreType.DMA,
            pltpu.SemaphoreType.DMA,
        ],
        compiler_params=pltpu.CompilerParams(use_tc_tiling_on_sc=False),
    )
    def k(sd_hbm, t1_hbm, t2_hbm, h_hbm,
          acc0_hbm, acc1_hbm, den0_hbm, den1_hbm,
          idx0, idx1, sx0, sx1, gs0, gs1, gd0, gd1, hg0, hg1,
          out_sh, den_sh, si0, si1, sg0, sg1, ss0, ss1):
        c = lax.axis_index("c")
        s = lax.axis_index("s")
        wid = c * NS + s
        idxb, sidx = [idx0, idx1], [sx0, sx1]
        gs, gd, hg = [gs0, gs1], [gd0, gd1], [hg0, hg1]
        semi, semg, sems = [si0, si1], [sg0, sg1], [ss0, ss1]
        zero16 = jnp.zeros((LANES,), jnp.float32)

        def zfill(i, _):
            gs0[i, :] = zero16
            for j in range(FEAT // LANES):
                hg0[i, pl.ds(j * LANES, LANES)] = zero16
            return 0
        lax.fori_loop(0, ZROW, zfill, 0)

        def zcopy(j, _):
            pltpu.sync_copy(hg0.at[pl.ds(0, ZROW)],
                            out_sh.at[pl.ds(s * rows_pt + j * ZROW, ZROW)])
            pltpu.sync_copy(gs0.at[pl.ds(0, ZROW)],
                            den_sh.at[pl.ds(s * rows_pt + j * ZROW, ZROW)])
            return 0
        lax.fori_loop(0, rows_pt // ZROW, zcopy, 0)
        plsc.subcore_barrier()

        def issue_idx(p, t):
            gc = wid * nchunks + jnp.minimum(t, nchunks - 1)
            pltpu.async_copy(sd_hbm.at[gc], idxb[p], semi[p])

        def wait_idx(p):
            pltpu.make_async_copy(sd_hbm.at[0], idxb[p], semi[p]).wait()

        def issue_g(p, t):
            wait_idx(p)
            pltpu.async_copy(t1_hbm.at[idxb[p].at[0]], gs[p], semg[p])
            pltpu.async_copy(t2_hbm.at[idxb[p].at[1]], gd[p], semg[p])
            pltpu.async_copy(h_hbm.at[idxb[p].at[0]], hg[p], semg[p])

        def wait_g(p):
            pltpu.make_async_copy(t1_hbm.at[idxb[p].at[0]], gs[p],
                                  semg[p]).wait()
            pltpu.make_async_copy(t2_hbm.at[idxb[p].at[1]], gd[p],
                                  semg[p]).wait()
            pltpu.make_async_copy(h_hbm.at[idxb[p].at[0]], hg[p],
                                  semg[p]).wait()

        def copy_sidx(p):
            for j in range(CHUNK // LANES):
                sidx[p][pl.ds(j * LANES, LANES)] = \
                    idxb[p][1, pl.ds(j * LANES, LANES)]

        def compute(p):
            gsp, gdp, hgp = gs[p], gd[p], hg[p]

            def ex_body(i, _):
                for u in range(8):
                    e = i * 8 + u
                    a = gsp[e, :] + gdp[e, :]
                    a = jnp.maximum(a, 0.2 * a)
                    gsp[e, :] = jnp.exp(a)
                return 0
            lax.fori_loop(0, CHUNK // 8, ex_body, 0)

            def scale_body(i, _):
                for u in range(4):
                    e = i * 4 + u
                    ex = gsp[e, :]
                    for hh in range(HEADS):
                        seg = hgp[e, pl.ds(hh * LANES, LANES)]
                        hgp[e, pl.ds(hh * LANES, LANES)] = (
                            seg * _splat(ex, hh))
                return 0
            lax.fori_loop(0, CHUNK // 4, scale_body, 0)

        def scatter(p):
            pltpu.async_copy(gs[p], den_sh.at[sidx[p]], sems[p], add=True)
            pltpu.async_copy(hg[p], out_sh.at[sidx[p]], sems[p], add=True)

        def wait_s(p):
            pltpu.make_async_copy(gs[p], den_sh.at[sidx[p]],
                                  sems[p]).wait()
            pltpu.make_async_copy(hg[p], out_sh.at[sidx[p]],
                                  sems[p]).wait()

        issue_idx(0, 0)
        issue_g(0, 0)
        issue_idx(1, 1)

        wait_g(0)
        issue_g(1, 1)
        copy_sidx(0)
        issue_idx(0, 2)
        compute(0)
        scatter(0)
        wait_g(1)
        copy_sidx(1)
        issue_idx(1, 3)
        wait_s(0)
        issue_g(0, 2)
        compute(1)
        scatter(1)

        def pair_body(g, _):
            wait_g(0)
            wait_s(1)
            issue_g(1, 2 * g + 1)
            copy_sidx(0)
            issue_idx(0, 2 * g + 2)
            compute(0)
            scatter(0)
            wait_g(1)
            wait_s(0)
            issue_g(0, 2 * g + 2)
            copy_sidx(1)
            issue_idx(1, 2 * g + 3)
            compute(1)
            scatter(1)
            return 0
        lax.fori_loop(1, (nchunks - 1) // 2, pair_body, 0)

        wait_g(0)
        wait_s(1)
        copy_sidx(0)
        compute(0)
        scatter(0)
        wait_s(0)
        wait_idx(1)
        plsc.subcore_barrier()

        @pl.when(c == 0)
        def _():
            pltpu.sync_copy(out_sh.at[pl.ds(s * rows_pt, rows_pt)],
                            acc0_hbm.at[pl.ds(s * rows_pt, rows_pt)])
            pltpu.sync_copy(den_sh.at[pl.ds(s * rows_pt, rows_pt)],
                            den0_hbm.at[pl.ds(s * rows_pt, rows_pt)])

        @pl.when(c == 1)
        def _():
            pltpu.sync_copy(out_sh.at[pl.ds(s * rows_pt, rows_pt)],
                            acc1_hbm.at[pl.ds(s * rows_pt, rows_pt)])
            pltpu.sync_copy(den_sh.at[pl.ds(s * rows_pt, rows_pt)],
                            den1_hbm.at[pl.ds(s * rows_pt, rows_pt)])

    return k(sd, t1, t2, h)


def kernel(x, edge_index, W1, a1_src, a1_dst, b1, W2, a2_src, a2_dst, b2):
    n, d = x.shape
    assert d == FEAT
    e = edge_index.shape[1]
    e_tot = e + n

    npad = ((n + LANES) + 1280 - 1) // 1280 * 1280
    br = npad // 8
    nchunks = -(-e_tot // (NW * CHUNK))
    if nchunks % 2 == 0 or nchunks < 3:
        nchunks += max(3 - nchunks, 1)
    we = nchunks * CHUNK
    e_pad = NW * we

    loop = jnp.arange(n, dtype=jnp.int32)
    junk = jnp.full((e_pad - e_tot,), n, dtype=jnp.int32)
    src = jnp.concatenate([edge_index[0], loop, junk])
    dst = jnp.concatenate([edge_index[1], loop, junk])
    sd = jnp.stack([src.reshape(-1, CHUNK), dst.reshape(-1, CHUNK)], axis=1)

    x_pad = jnp.zeros((npad, d), jnp.float32).at[:n].set(x)
    a32_1 = _build_a32(a1_src, a1_dst)
    a32_2 = _build_a32(a2_src, a2_dst)
    cols = jnp.arange(FEAT)
    rep = jnp.zeros((16, FEAT), jnp.float32).at[cols // OUT_CH, cols].set(1.0)

    h1, t1a, t1b = _tc_head(x_pad, W1, a32_1, npad, br)
    acc10, acc11, den10, den11 = _sc_layer(sd, t1a, t1b, h1, npad,
                                           e_pad, we, nchunks)
    h2, t2a, t2b = _tc_mid(acc10, acc11, den10, den11, rep, b1, W2, a32_2,
                           npad, br)
    acc20, acc21, den20, den21 = _sc_layer(sd, t2a, t2b, h2, npad,
                                           e_pad, we, nchunks)
    out = _tc_tail(acc20, acc21, den20, den21, rep, b2, npad, br)
    return out[:n]


# speedup vs baseline: 1.0073x; 1.0073x over previous
import functools

import jax
import jax.numpy as jnp
from jax import lax
from jax.experimental import pallas as pl
from jax.experimental.pallas import tpu as pltpu
from jax.experimental.pallas import tpu_sc as plsc

HEADS = 8
OUT_CH = 16
FEAT = HEADS * OUT_CH
LANES = 16
NC = 2
NS = 16
NW = NC * NS
CHUNK = 112
ZROW = 64

_GD = lax.GatherDimensionNumbers(
    offset_dims=(), collapsed_slice_dims=(0,), start_index_map=(0,))


def _splat(v, h):
    idx = jnp.full((LANES, 1), h, dtype=jnp.int32)
    return lax.gather(v, idx, _GD, (1,),
                      mode=lax.GatherScatterMode.PROMISE_IN_BOUNDS)


def _build_a32(a_src, a_dst):
    rows = jnp.arange(FEAT)
    head = rows // OUT_CH
    a = jnp.zeros((FEAT, 2 * HEADS), jnp.float32)
    a = a.at[rows, head].set(a_src.reshape(-1))
    a = a.at[rows, HEADS + head].set(a_dst.reshape(-1))
    swap = jnp.concatenate([a[:, HEADS:], a[:, :HEADS]], axis=1)
    return jnp.concatenate([a, swap], axis=1)


def _tc_head_body(x_ref, w_ref, a_ref, h_ref, t1_ref, t2_ref):
    h = jnp.dot(x_ref[...], w_ref[...], preferred_element_type=jnp.float32)
    h_ref[...] = h
    t = jnp.dot(h, a_ref[...], preferred_element_type=jnp.float32)
    t1_ref[...] = t[:, :16]
    t2_ref[...] = t[:, 16:]


def _tc_mid_body(a0_ref, a1_ref, d0_ref, d1_ref, rep_ref, b_ref, w_ref,
                 a_ref, h_ref, t1_ref, t2_ref):
    dinv = 1.0 / (d0_ref[...] + d1_ref[...] + 1e-16)
    scale = jnp.dot(dinv, rep_ref[...], preferred_element_type=jnp.float32)
    x = jnp.maximum((a0_ref[...] + a1_ref[...]) * scale + b_ref[...], 0.0)
    h = jnp.dot(x, w_ref[...], preferred_element_type=jnp.float32)
    h_ref[...] = h
    t = jnp.dot(h, a_ref[...], preferred_element_type=jnp.float32)
    t1_ref[...] = t[:, :16]
    t2_ref[...] = t[:, 16:]


def _tc_tail_body(a0_ref, a1_ref, d0_ref, d1_ref, rep_ref, b_ref, o_ref):
    dinv = 1.0 / (d0_ref[...] + d1_ref[...] + 1e-16)
    scale = jnp.dot(dinv, rep_ref[...], preferred_element_type=jnp.float32)
    o_ref[...] = (a0_ref[...] + a1_ref[...]) * scale + b_ref[...]


def _tc_head(x_pad, w, a32, npad, br):
    grid = npad // br
    return pl.pallas_call(
        _tc_head_body,
        grid=(grid,),
        in_specs=[
            pl.BlockSpec((br, FEAT), lambda i: (i, 0)),
            pl.BlockSpec((FEAT, FEAT), lambda i: (0, 0)),
            pl.BlockSpec((FEAT, 32), lambda i: (0, 0)),
        ],
        out_specs=[
            pl.BlockSpec((br, FEAT), lambda i: (i, 0)),
            pl.BlockSpec((br, 16), lambda i: (i, 0)),
            pl.BlockSpec((br, 16), lambda i: (i, 0)),
        ],
        out_shape=[
            jax.ShapeDtypeStruct((npad, FEAT), jnp.float32),
            jax.ShapeDtypeStruct((npad, 16), jnp.float32),
            jax.ShapeDtypeStruct((npad, 16), jnp.float32),
        ],
    )(x_pad, w, a32)


def _tc_mid(acc0, acc1, d0, d1, rep, b, w, a32, npad, br):
    grid = npad // br
    return pl.pallas_call(
        _tc_mid_body,
        grid=(grid,),
        in_specs=[
            pl.BlockSpec((br, FEAT), lambda i: (i, 0)),
            pl.BlockSpec((br, FEAT), lambda i: (i, 0)),
            pl.BlockSpec((br, 16), lambda i: (i, 0)),
            pl.BlockSpec((br, 16), lambda i: (i, 0)),
            pl.BlockSpec((16, FEAT), lambda i: (0, 0)),
            pl.BlockSpec((1, FEAT), lambda i: (0, 0)),
            pl.BlockSpec((FEAT, FEAT), lambda i: (0, 0)),
            pl.BlockSpec((FEAT, 32), lambda i: (0, 0)),
        ],
        out_specs=[
            pl.BlockSpec((br, FEAT), lambda i: (i, 0)),
            pl.BlockSpec((br, 16), lambda i: (i, 0)),
            pl.BlockSpec((br, 16), lambda i: (i, 0)),
        ],
        out_shape=[
            jax.ShapeDtypeStruct((npad, FEAT), jnp.float32),
            jax.ShapeDtypeStruct((npad, 16), jnp.float32),
            jax.ShapeDtypeStruct((npad, 16), jnp.float32),
        ],
    )(acc0, acc1, d0, d1, rep, b.reshape(1, FEAT), w, a32)


def _tc_tail(acc0, acc1, d0, d1, rep, b, npad, br):
    grid = npad // br
    return pl.pallas_call(
        _tc_tail_body,
        grid=(grid,),
        in_specs=[
            pl.BlockSpec((br, FEAT), lambda i: (i, 0)),
            pl.BlockSpec((br, FEAT), lambda i: (i, 0)),
            pl.BlockSpec((br, 16), lambda i: (i, 0)),
            pl.BlockSpec((br, 16), lambda i: (i, 0)),
            pl.BlockSpec((16, FEAT), lambda i: (0, 0)),
            pl.BlockSpec((1, FEAT), lambda i: (0, 0)),
        ],
        out_specs=pl.BlockSpec((br, FEAT), lambda i: (i, 0)),
        out_shape=jax.ShapeDtypeStruct((npad, FEAT), jnp.float32),
    )(acc0, acc1, d0, d1, rep, b.reshape(1, FEAT))


def _sc_layer(sd, t1, t2, h, npad, e_pad, we, nchunks):
    rows_pt = npad // NS
    mesh = plsc.VectorSubcoreMesh(
        core_axis_name="c", subcore_axis_name="s", num_cores=NC,
        num_subcores=NS)

    @functools.partial(
        pl.kernel,
        out_type=[
            jax.ShapeDtypeStruct((npad, FEAT), jnp.float32),
            jax.ShapeDtypeStruct((npad, FEAT), jnp.float32),
            jax.ShapeDtypeStruct((npad, 16), jnp.float32),
            jax.ShapeDtypeStruct((npad, 16), jnp.float32),
        ],
        mesh=mesh,
        scratch_types=[
            pltpu.VMEM((2, CHUNK), jnp.int32),
            pltpu.VMEM((2, CHUNK), jnp.int32),
            pltpu.VMEM((CHUNK,), jnp.int32),
            pltpu.VMEM((CHUNK,), jnp.int32),
            pltpu.VMEM((CHUNK, 16), jnp.float32),
            pltpu.VMEM((CHUNK, 16), jnp.float32),
            pltpu.VMEM((CHUNK, 16), jnp.float32),
            pltpu.VMEM((CHUNK, 16), jnp.float32),
            pltpu.VMEM((CHUNK, FEAT), jnp.float32),
            pltpu.VMEM((CHUNK, FEAT), jnp.float32),
            pltpu.VMEM_SHARED((npad, FEAT), jnp.float32),
            pltpu.VMEM_SHARED((npad, 16), jnp.float32),
            pltpu.SemaphoreType.DMA,
            pltpu.SemaphoreType.DMA,
            pltpu.SemaphoreType.DMA,
            pltpu.SemaphoreType.DMA,
            pltpu.SemaphoreType.DMA,
            pltpu.SemaphoreType.DMA,
        ],
        compiler_params=pltpu.CompilerParams(use_tc_tiling_on_sc=False),
    )
    def k(sd_hbm, t1_hbm, t2_hbm, h_hbm,
          acc0_hbm, acc1_hbm, den0_hbm, den1_hbm,
          idx0, idx1, sx0, sx1, gs0, gs1, gd0, gd1, hg0, hg1,
          out_sh, den_sh, si0, si1, sg0, sg1, ss0, ss1):
        c = lax.axis_index("c")
        s = lax.axis_index("s")
        wid = c * NS + s
        idxb, sidx = [idx0, idx1], [sx0, sx1]
        gs, gd, hg = [gs0, gs1], [gd0, gd1], [hg0, hg1]
        semi, semg, sems = [si0, si1], [sg0, sg1], [ss0, ss1]
        zero16 = jnp.zeros((LANES,), jnp.float32)

        def zfill(i, _):
            gs0[i, :] = zero16
            for j in range(FEAT // LANES):
                hg0[i, pl.ds(j * LANES, LANES)] = zero16
            return 0
        lax.fori_loop(0, ZROW, zfill, 0)

        def zcopy(j, _):
            pltpu.sync_copy(hg0.at[pl.ds(0, ZROW)],
                            out_sh.at[pl.ds(s * rows_pt + j * ZROW, ZROW)])
            pltpu.sync_copy(gs0.at[pl.ds(0, ZROW)],
                            den_sh.at[pl.ds(s * rows_pt + j * ZROW, ZROW)])
            return 0
        lax.fori_loop(0, rows_pt // ZROW, zcopy, 0)
        plsc.subcore_barrier()

        def issue_idx(p, t):
            gc = wid * nchunks + jnp.minimum(t, nchunks - 1)
            pltpu.async_copy(sd_hbm.at[gc], idxb[p], semi[p])

        def wait_idx(p):
            pltpu.make_async_copy(sd_hbm.at[0], idxb[p], semi[p]).wait()

        def issue_g(p, t):
            wait_idx(p)
            pltpu.async_copy(t1_hbm.at[idxb[p].at[0]], gs[p], semg[p])
            pltpu.async_copy(t2_hbm.at[idxb[p].at[1]], gd[p], semg[p])
            pltpu.async_copy(h_hbm.at[idxb[p].at[0]], hg[p], semg[p])

        def wait_g(p):
            pltpu.make_async_copy(t1_hbm.at[idxb[p].at[0]], gs[p],
                                  semg[p]).wait()
            pltpu.make_async_copy(t2_hbm.at[idxb[p].at[1]], gd[p],
                                  semg[p]).wait()
            pltpu.make_async_copy(h_hbm.at[idxb[p].at[0]], hg[p],
                                  semg[p]).wait()

        def copy_sidx(p):
            for j in range(CHUNK // LANES):
                sidx[p][pl.ds(j * LANES, LANES)] = \
                    idxb[p][1, pl.ds(j * LANES, LANES)]

        def compute(p):
            gsp, gdp, hgp = gs[p], gd[p], hg[p]

            def ex_body(i, _):
                for u in range(8):
                    e = i * 8 + u
                    a = gsp[e, :] + gdp[e, :]
                    a = jnp.maximum(a, 0.2 * a)
                    gsp[e, :] = jnp.exp(a)
                return 0
            lax.fori_loop(0, CHUNK // 8, ex_body, 0)

            def scale_body(i, _):
                for u in range(2):
                    e = i * 2 + u
                    ex = gsp[e, :]
                    for hh in range(HEADS):
                        seg = hgp[e, pl.ds(hh * LANES, LANES)]
                        hgp[e, pl.ds(hh * LANES, LANES)] = (
                            seg * _splat(ex, hh))
                return 0
            lax.fori_loop(0, CHUNK // 2, scale_body, 0)

        def scatter(p):
            pltpu.async_copy(gs[p], den_sh.at[sidx[p]], sems[p], add=True)
            pltpu.async_copy(hg[p], out_sh.at[sidx[p]], sems[p], add=True)

        def wait_s(p):
            pltpu.make_async_copy(gs[p], den_sh.at[sidx[p]],
                                  sems[p]).wait()
            pltpu.make_async_copy(hg[p], out_sh.at[sidx[p]],
                                  sems[p]).wait()

        issue_idx(0, 0)
        issue_g(0, 0)
        issue_idx(1, 1)

        wait_g(0)
        issue_g(1, 1)
        copy_sidx(0)
        issue_idx(0, 2)
        compute(0)
        scatter(0)
        wait_g(1)
        copy_sidx(1)
        issue_idx(1, 3)
        wait_s(0)
        issue_g(0, 2)
        compute(1)
        scatter(1)

        def pair_body(g, _):
            wait_g(0)
            wait_s(1)
            issue_g(1, 2 * g + 1)
            copy_sidx(0)
            issue_idx(0, 2 * g + 2)
            compute(0)
            scatter(0)
            wait_g(1)
            wait_s(0)
            issue_g(0, 2 * g + 2)
            copy_sidx(1)
            issue_idx(1, 2 * g + 3)
            compute(1)
            scatter(1)
            return 0
        lax.fori_loop(1, (nchunks - 1) // 2, pair_body, 0)

        wait_g(0)
        wait_s(1)
        copy_sidx(0)
        compute(0)
        scatter(0)
        wait_s(0)
        wait_idx(1)
        plsc.subcore_barrier()

        @pl.when(c == 0)
        def _():
            pltpu.sync_copy(out_sh.at[pl.ds(s * rows_pt, rows_pt)],
                            acc0_hbm.at[pl.ds(s * rows_pt, rows_pt)])
            pltpu.sync_copy(den_sh.at[pl.ds(s * rows_pt, rows_pt)],
                            den0_hbm.at[pl.ds(s * rows_pt, rows_pt)])

        @pl.when(c == 1)
        def _():
            pltpu.sync_copy(out_sh.at[pl.ds(s * rows_pt, rows_pt)],
                            acc1_hbm.at[pl.ds(s * rows_pt, rows_pt)])
            pltpu.sync_copy(den_sh.at[pl.ds(s * rows_pt, rows_pt)],
                            den1_hbm.at[pl.ds(s * rows_pt, rows_pt)])

    return k(sd, t1, t2, h)


def kernel(x, edge_index, W1, a1_src, a1_dst, b1, W2, a2_src, a2_dst, b2):
    n, d = x.shape
    assert d == FEAT
    e = edge_index.shape[1]
    e_tot = e + n

    npad = ((n + LANES) + 1280 - 1) // 1280 * 1280
    br = npad // 8
    nchunks = -(-e_tot // (NW * CHUNK))
    if nchunks % 2 == 0 or nchunks < 3:
        nchunks += max(3 - nchunks, 1)
    we = nchunks * CHUNK
    e_pad = NW * we

    loop = jnp.arange(n, dtype=jnp.int32)
    junk = jnp.full((e_pad - e_tot,), n, dtype=jnp.int32)
    src = jnp.concatenate([edge_index[0], loop, junk])
    dst = jnp.concatenate([edge_index[1], loop, junk])
    sd = jnp.stack([src.reshape(-1, CHUNK), dst.reshape(-1, CHUNK)], axis=1)

    x_pad = jnp.zeros((npad, d), jnp.float32).at[:n].set(x)
    a32_1 = _build_a32(a1_src, a1_dst)
    a32_2 = _build_a32(a2_src, a2_dst)
    cols = jnp.arange(FEAT)
    rep = jnp.zeros((16, FEAT), jnp.float32).at[cols // OUT_CH, cols].set(1.0)

    h1, t1a, t1b = _tc_head(x_pad, W1, a32_1, npad, br)
    acc10, acc11, den10, den11 = _sc_layer(sd, t1a, t1b, h1, npad,
                                           e_pad, we, nchunks)
    h2, t2a, t2b = _tc_mid(acc10, acc11, den10, den11, rep, b1, W2, a32_2,
                           npad, br)
    acc20, acc21, den20, den21 = _sc_layer(sd, t2a, t2b, h2, npad,
                                           e_pad, we, nchunks)
    out = _tc_tail(acc20, acc21, den20, den21, rep, b2, npad, br)
    return out[:n]
